# Initial kernel scaffold; baseline (speedup 1.0000x reference)
import jax
import jax.numpy as jnp
from jax.experimental import pallas as pl


def kernel(x, edge_index, W, b):
    raise NotImplementedError("write your pallas kernel here")


# speedup vs baseline: 30.6872x; 30.6872x over previous
import functools

import jax
import jax.numpy as jnp
from jax import lax
from jax.experimental import pallas as pl
from jax.experimental.pallas import tpu as pltpu
from jax.experimental.pallas import tpu_sc as plsc

N = 10000
E = 320000
D = 128
NPAD = 10240
NC, NS, L = 2, 16, 16
NW = NC * NS
CH = 80
EPAD = NW * CH * 128
RPT = NPAD // NS


def _mesh():
    return plsc.VectorSubcoreMesh(core_axis_name="c", subcore_axis_name="s")


def _sc_hist(dst2d, zeros1d):
    @functools.partial(
        pl.kernel,
        out_type=jax.ShapeDtypeStruct((NC, NPAD), jnp.float32),
        mesh=_mesh(),
        scratch_types=[
            pltpu.VMEM((CH, 128), jnp.int32),
            pltpu.VMEM((128,), jnp.float32),
            pltpu.VMEM_SHARED((NPAD,), jnp.float32),
        ],
    )
    def k(dst_hbm, z_hbm, out_hbm, dst_v, ones_v, hist_sh):
        c = lax.axis_index("c")
        s = lax.axis_index("s")
        w = c * NS + s
        pltpu.sync_copy(dst_hbm.at[pl.ds(w * CH, CH)], dst_v)
        pltpu.sync_copy(z_hbm.at[pl.ds(s * RPT, RPT)],
                        hist_sh.at[pl.ds(s * RPT, RPT)])

        @pl.loop(0, 8)
        def _(i):
            ones_v[pl.ds(i * L, L)] = jnp.ones((L,), jnp.float32)

        plsc.subcore_barrier()

        @pl.loop(0, CH)
        def _(j):
            pltpu.sync_copy(ones_v, hist_sh.at[dst_v.at[j]], add=True)

        plsc.subcore_barrier()
        pltpu.sync_copy(hist_sh.at[pl.ds(s * RPT, RPT)],
                        out_hbm.at[c, pl.ds(s * RPT, RPT)])

    return k(dst2d, zeros1d)


def _sc_msg(y, src2d, dst2d, zeros2d):
    @functools.partial(
        pl.kernel,
        out_type=jax.ShapeDtypeStruct((NC, NPAD, D), jnp.float32),
        mesh=_mesh(),
        scratch_types=[
            pltpu.VMEM((CH, 128), jnp.int32),
            pltpu.VMEM((CH, 128), jnp.int32),
            pltpu.VMEM((128, D), jnp.float32),
            pltpu.VMEM_SHARED((NPAD, D), jnp.float32),
        ],
    )
    def k(y_hbm, src_hbm, dst_hbm, z_hbm, out_hbm, src_v, dst_v, buf, accum):
        c = lax.axis_index("c")
        s = lax.axis_index("s")
        w = c * NS + s
        pltpu.sync_copy(src_hbm.at[pl.ds(w * CH, CH)], src_v)
        pltpu.sync_copy(dst_hbm.at[pl.ds(w * CH, CH)], dst_v)
        pltpu.sync_copy(z_hbm.at[pl.ds(s * RPT, RPT)],
                        accum.at[pl.ds(s * RPT, RPT)])
        plsc.subcore_barrier()

        @pl.loop(0, CH)
        def _(j):
            pltpu.sync_copy(y_hbm.at[src_v.at[j]], buf)
            pltpu.sync_copy(buf, accum.at[dst_v.at[j]], add=True)

        plsc.subcore_barrier()
        pltpu.sync_copy(accum.at[pl.ds(s * RPT, RPT)],
                        out_hbm.at[c, pl.ds(s * RPT, RPT)])

    return k(y, src2d, dst2d, zeros2d)


def _tc_mm(x_pad, w, hist_t):
    def body(x_ref, w_ref, h_ref, y_ref, xw_ref):
        deg = h_ref[:, 0:1] + h_ref[:, 1:2] + 1.0
        dis = lax.rsqrt(deg)
        xw = jnp.dot(x_ref[...], w_ref[...],
                     preferred_element_type=jnp.float32,
                     precision=lax.Precision.HIGHEST)
        xw_ref[...] = xw
        y_ref[...] = xw * dis

    return pl.pallas_call(
        body,
        out_shape=(jax.ShapeDtypeStruct((NPAD, D), jnp.float32),
                   jax.ShapeDtypeStruct((NPAD, D), jnp.float32)),
    )(x_pad, w, hist_t)


def _tc_final(partials, hist_t, xw, b2d):
    def body(p_ref, h_ref, xw_ref, b_ref, o_ref):
        deg = h_ref[:, 0:1] + h_ref[:, 1:2] + 1.0
        dis = lax.rsqrt(deg)
        agg = p_ref[0] + p_ref[1]
        res = agg * dis + xw_ref[...] * (dis * dis) + b_ref[...]
        o_ref[...] = res[:N, :]

    return pl.pallas_call(
        body,
        out_shape=jax.ShapeDtypeStruct((N, D), jnp.float32),
    )(partials, hist_t, xw, b2d)


def kernel(x, edge_index, W, b):
    src = edge_index[0]
    dst = edge_index[1]
    pad = EPAD - E
    pad_idx = N + (jnp.arange(pad, dtype=jnp.int32) % (NPAD - N))
    src_p = jnp.concatenate([src, pad_idx]).reshape(NW * CH, 128)
    dst_p = jnp.concatenate([dst, pad_idx]).reshape(NW * CH, 128)
    x_pad = jnp.pad(x, ((0, NPAD - N), (0, 0)))
    z1 = jnp.zeros((NPAD,), jnp.float32)
    z2 = jnp.zeros((NPAD, D), jnp.float32)

    hist = _sc_hist(dst_p, z1)
    hist_t = hist.T
    y, xw = _tc_mm(x_pad, W, hist_t)
    part = _sc_msg(y, src_p, dst_p, z2)
    return _tc_final(part, hist_t, xw, b.reshape(1, D))


# speedup vs baseline: 43.7832x; 1.4268x over previous
import functools

import jax
import jax.numpy as jnp
from jax import lax
from jax.experimental import pallas as pl
from jax.experimental.pallas import tpu as pltpu
from jax.experimental.pallas import tpu_sc as plsc

N = 10000
E = 320000
D = 128
NPAD = 10240
NC, NS, L = 2, 16, 16
NW = NC * NS
CH = 80
EPAD = NW * CH * 128
RPT = NPAD // NS


def _mesh():
    return plsc.VectorSubcoreMesh(core_axis_name="c", subcore_axis_name="s")


HIST_WIN = 8


def _sc_hist(dst2d):
    @functools.partial(
        pl.kernel,
        out_type=jax.ShapeDtypeStruct((NC, NPAD), jnp.float32),
        mesh=_mesh(),
        scratch_types=[
            pltpu.VMEM((CH, 128), jnp.int32),
            pltpu.VMEM((128,), jnp.float32),
            pltpu.VMEM((RPT,), jnp.float32),
            pltpu.VMEM_SHARED((NPAD,), jnp.float32),
            pltpu.SemaphoreType.DMA,
        ],
    )
    def k(dst_hbm, out_hbm, dst_v, ones_v, zv, hist_sh, sem):
        c = lax.axis_index("c")
        s = lax.axis_index("s")
        w = c * NS + s
        pltpu.sync_copy(dst_hbm.at[pl.ds(w * CH, CH)], dst_v)

        @pl.loop(0, 8)
        def _(i):
            ones_v[pl.ds(i * L, L)] = jnp.ones((L,), jnp.float32)

        @pl.loop(0, RPT // L)
        def _(i):
            zv[pl.ds(i * L, L)] = jnp.zeros((L,), jnp.float32)

        pltpu.sync_copy(zv, hist_sh.at[pl.ds(s * RPT, RPT)])
        plsc.subcore_barrier()

        @pl.loop(0, CH)
        def _(j):
            pltpu.async_copy(ones_v, hist_sh.at[dst_v.at[j]], sem, add=True)

            @pl.when(j >= HIST_WIN)
            def _():
                pltpu.make_async_copy(
                    ones_v, hist_sh.at[dst_v.at[j - HIST_WIN]], sem).wait()

        @pl.loop(CH - HIST_WIN, CH)
        def _(j):
            pltpu.make_async_copy(ones_v, hist_sh.at[dst_v.at[j]], sem).wait()

        plsc.subcore_barrier()
        pltpu.sync_copy(hist_sh.at[pl.ds(s * RPT, RPT)],
                        out_hbm.at[c, pl.ds(s * RPT, RPT)])

    return k(dst2d)


NQ = 5
QC = CH // NQ


def _sc_msg(y, src2d, dst2d):
    @functools.partial(
        pl.kernel,
        out_type=jax.ShapeDtypeStruct((NC, NPAD, D), jnp.float32),
        mesh=_mesh(),
        scratch_types=[
            [pltpu.VMEM((QC, 128), jnp.int32)] * 2,
            [pltpu.VMEM((QC, 128), jnp.int32)] * 2,
            [pltpu.VMEM((128, D), jnp.float32)] * 2,
            pltpu.VMEM_SHARED((NPAD, D), jnp.float32),
            [pltpu.SemaphoreType.DMA] * 2,
            pltpu.SemaphoreType.DMA,
        ],
    )
    def k(y_hbm, src_hbm, dst_hbm, out_hbm, srcq, dstq, bufs, accum,
          gsems, isem):
        c = lax.axis_index("c")
        s = lax.axis_index("s")
        w = c * NS + s
        base = w * CH

        pltpu.async_copy(src_hbm.at[pl.ds(base, QC)], srcq[0], isem)
        pltpu.async_copy(dst_hbm.at[pl.ds(base, QC)], dstq[0], isem)

        @pl.loop(0, 128)
        def _(i):
            @pl.loop(0, D // L)
            def _(q):
                bufs[0][i, pl.ds(q * L, L)] = jnp.zeros((L,), jnp.float32)

        for r in range(RPT // 128):
            pltpu.sync_copy(bufs[0], accum.at[pl.ds(s * RPT + r * 128, 128)])

        pltpu.make_async_copy(src_hbm.at[pl.ds(base, QC)], srcq[0],
                              isem).wait()
        pltpu.make_async_copy(dst_hbm.at[pl.ds(base, QC)], dstq[0],
                              isem).wait()
        plsc.subcore_barrier()

        def g_start(i, sl, b):
            pltpu.async_copy(y_hbm.at[srcq[sl].at[i]], bufs[b], gsems[b])

        def g_wait(i, sl, b):
            pltpu.make_async_copy(y_hbm.at[srcq[sl].at[i]], bufs[b],
                                  gsems[b]).wait()

        for q in range(NQ):
            sl = q % 2
            if q < NQ - 1:
                nb = base + (q + 1) * QC
                pltpu.async_copy(src_hbm.at[pl.ds(nb, QC)], srcq[1 - sl],
                                 isem)
                pltpu.async_copy(dst_hbm.at[pl.ds(nb, QC)], dstq[1 - sl],
                                 isem)

            for b in range(2):
                g_start(b, sl, b)

            @pl.loop(0, QC - 2, step=2)
            def _(i):
                for b in range(2):
                    ii = i + b
                    g_wait(ii, sl, b)
                    pltpu.sync_copy(bufs[b], accum.at[dstq[sl].at[ii]],
                                    add=True)
                    g_start(ii + 2, sl, b)

            for b in range(2):
                ii = QC - 2 + b
                g_wait(ii, sl, b)
                pltpu.sync_copy(bufs[b], accum.at[dstq[sl].at[ii]], add=True)

            if q < NQ - 1:
                nb = base + (q + 1) * QC
                pltpu.make_async_copy(src_hbm.at[pl.ds(nb, QC)],
                                      srcq[1 - sl], isem).wait()
                pltpu.make_async_copy(dst_hbm.at[pl.ds(nb, QC)],
                                      dstq[1 - sl], isem).wait()

        plsc.subcore_barrier()
        pltpu.sync_copy(accum.at[pl.ds(s * RPT, RPT)],
                        out_hbm.at[c, pl.ds(s * RPT, RPT)])

    return k(y, src2d, dst2d)


def _tc_mm(x_pad, w, hist_t):
    def body(x_ref, w_ref, h_ref, y_ref, xw_ref):
        deg = h_ref[:, 0:1] + h_ref[:, 1:2] + 1.0
        dis = lax.rsqrt(deg)
        xw = jnp.dot(x_ref[...], w_ref[...],
                     preferred_element_type=jnp.float32,
                     precision=lax.Precision.HIGHEST)
        xw_ref[...] = xw
        y_ref[...] = xw * dis

    return pl.pallas_call(
        body,
        out_shape=(jax.ShapeDtypeStruct((NPAD, D), jnp.float32),
                   jax.ShapeDtypeStruct((NPAD, D), jnp.float32)),
    )(x_pad, w, hist_t)


def _tc_final(partials, hist_t, xw, b2d):
    def body(p_ref, h_ref, xw_ref, b_ref, o_ref):
        deg = h_ref[:, 0:1] + h_ref[:, 1:2] + 1.0
        dis = lax.rsqrt(deg)
        agg = p_ref[0] + p_ref[1]
        res = agg * dis + xw_ref[...] * (dis * dis) + b_ref[...]
        o_ref[...] = res[:N, :]

    return pl.pallas_call(
        body,
        out_shape=jax.ShapeDtypeStruct((N, D), jnp.float32),
    )(partials, hist_t, xw, b2d)


def kernel(x, edge_index, W, b):
    src = edge_index[0]
    dst = edge_index[1]
    pad = EPAD - E
    pad_idx = N + (jnp.arange(pad, dtype=jnp.int32) % (NPAD - N))
    src_p = jnp.concatenate([src, pad_idx]).reshape(NW * CH, 128)
    dst_p = jnp.concatenate([dst, pad_idx]).reshape(NW * CH, 128)
    x_pad = jnp.pad(x, ((0, NPAD - N), (0, 0)))

    hist = _sc_hist(dst_p)
    hist_t = hist.T
    y, xw = _tc_mm(x_pad, W, hist_t)
    part = _sc_msg(y, src_p, dst_p)
    return _tc_final(part, hist_t, xw, b.reshape(1, D))
